# baseline (speedup 1.0000x reference)
import jax
import jax.numpy as jnp
from jax import lax
from jax.experimental import pallas as pl
from jax.experimental.pallas import tpu as pltpu
from jax.experimental.pallas import tpu_sc as plsc

N_S = 10000
N_A = 10000
E = 320000
D = 128
H = 128

NC = 2
NSUB = 16
NW = NC * NSUB
EPT = E // NW

W = D

KA = 80
NBLK_A = EPT // KA
KB = 2000
NBLK_B = EPT // KB
GRP = KA // 16

_F32 = jnp.float32
_I32 = jnp.int32
_BF16 = jnp.bfloat16
_ILV = None
_ILV = plsc.PackFormat.INTERLEAVED


def _tc_tables(es_ref, ea_ref, w1_ref, b1_ref, ts_ref, ta_ref):
    es = es_ref[...]
    ea = ea_ref[...]
    w1a = w1_ref[:D, :]
    w1b = w1_ref[D:, :]
    ts_ref[:, :D] = es
    ts_ref[:, D:] = jnp.dot(es, w1a, preferred_element_type=_F32) + b1_ref[...]
    ta_ref[:, :D] = ea
    ta_ref[:, D:] = jnp.dot(ea, w1b, preferred_element_type=_F32)


def _unpack2(word_vec):
    return plsc.unpack(plsc.bitcast(word_vec, _BF16), format=_ILV)


def _pass_a(ts_hbm, ta_hbm, sidx3_hbm, aidx3_hbm, w2_hbm, misc_hbm,
            ex_hbm, cost_hbm, partials_hbm,
            tsw0, tsw1, taw0, taw1, sidx_v, aidx_v, ex_v, cost_v,
            w2_v, misc_v, zeros_v, shared_seg, sem0, sem1, sem_sc):
    cid = lax.axis_index("c")
    sid = lax.axis_index("s")
    wid = sid * NC + cid

    pltpu.sync_copy(w2_hbm, w2_v)
    pltpu.sync_copy(misc_hbm, misc_v)
    pltpu.sync_copy(sidx3_hbm.at[wid], sidx_v)
    pltpu.sync_copy(aidx3_hbm.at[wid], aidx_v)


    def _zb(i, _):
        zeros_v[pl.ds(i * 16, 16)] = jnp.zeros((16,), _F32)
        return 0
    lax.fori_loop(0, 2048 // 16, _zb, 0)

    @pl.when(sid == 0)
    def _():
        for off, ln in ((0, 2048), (2048, 2048), (4096, 2048), (6144, 2048),
                        (8192, 1808)):
            pltpu.sync_copy(zeros_v.at[pl.ds(0, ln)],
                            shared_seg.at[pl.ds(off, ln)])

    plsc.subcore_barrier()

    iota16 = lax.iota(_I32, 16)
    z16 = jnp.zeros((16,), _F32)
    mv = misc_v[...]
    b2 = mv[0]
    invt = mv[1]

    tsbufs = (tsw0, tsw1)
    tabufs = (taw0, taw1)
    sems = (sem0, sem1)

    HK = KA // 2

    def fire(b, k):
        for tbl, idx, buf in ((ts_hbm, sidx_v, tsbufs[k]),
                              (ta_hbm, aidx_v, tabufs[k])):
            pltpu.async_copy(tbl.at[idx.at[b, pl.ds(0, HK)]],
                             buf.at[pl.ds(0, HK)], sems[k])
            pltpu.async_copy(tbl.at[idx.at[b, pl.ds(HK, HK)]],
                             buf.at[pl.ds(HK, HK)], sems[k])

    def drain(b, k):
        for tbl, idx, buf in ((ts_hbm, sidx_v, tsbufs[k]),
                              (ta_hbm, aidx_v, tabufs[k])):
            pltpu.make_async_copy(tbl.at[idx.at[b, pl.ds(0, HK)]],
                                  buf.at[pl.ds(0, HK)], sems[k]).wait()
            pltpu.make_async_copy(tbl.at[idx.at[b, pl.ds(HK, HK)]],
                                  buf.at[pl.ds(HK, HK)], sems[k]).wait()

    def compute(b, k):
        tsw = tsbufs[k]
        taw = tabufs[k]

        def gbody(g, _):
            rows = iota16 + g * 16
            zb = jnp.zeros((32,), _BF16)

            def jbody(jj, carry):
                acc_ss, acc_dot = carry
                w2a = w2_v[pl.ds(jj * 32, 16)]
                w2b = w2_v[pl.ds(jj * 32 + 16, 16)]
                for k2 in range(16):
                    colw = jnp.full((16,), jj * 16 + k2, _I32)
                    gts = plsc.bitcast(plsc.load_gather(tsw, [rows, colw]),
                                       _BF16)
                    gta = plsc.bitcast(plsc.load_gather(taw, [rows, colw]),
                                       _BF16)
                    d0, d1 = plsc.unpack(gts - gta, format=_ILV)
                    acc_ss = acc_ss + d0 * d0 + d1 * d1
                    colp = colw + (W // 2)
                    pts = plsc.bitcast(plsc.load_gather(tsw, [rows, colp]),
                                       _BF16)
                    pta = plsc.bitcast(plsc.load_gather(taw, [rows, colp]),
                                       _BF16)
                    h0, h1 = plsc.unpack(jnp.maximum(pts + pta, zb),
                                         format=_ILV)
                    if k2 < 8:
                        w20 = w2a[2 * k2]
                        w21 = w2a[2 * k2 + 1]
                    else:
                        w20 = w2b[2 * k2 - 16]
                        w21 = w2b[2 * k2 - 15]
                    acc_dot = (acc_dot + h0 * jnp.full((16,), w20, _F32)
                               + h1 * jnp.full((16,), w21, _F32))
                return acc_ss, acc_dot

            acc_ss, acc_dot = plsc.parallel_loop(
                0, W // 32, unroll=4, carry=(z16, z16))(jbody)

            ibits = plsc.bitcast(acc_ss, _I32)
            y = plsc.bitcast(jnp.int32(0x5F3759DF) - (ibits >> 1), _F32)
            for _ in range(3):
                y = y * (1.5 - 0.5 * acc_ss * y * y)
            cost = acc_ss * y

            gate = 1.0 / (1.0 + jnp.exp(-(acc_dot + b2)))
            costg = cost * gate
            ex = jnp.exp(-costg * invt)
            cost_v[b, pl.ds(g * 16, 16)] = costg
            ex_v[b, pl.ds(g * 16, 16)] = ex
            return 0

        lax.fori_loop(0, GRP, gbody, 0)
        pltpu.async_copy(ex_v.at[b], shared_seg.at[sidx_v.at[b]], sem_sc,
                         add=True)

    fire(0, 0)

    def pair_body(p, _):
        for k in (0, 1):
            b = p * 2 + k

            @pl.when(b + 1 < NBLK_A)
            def _():
                fire(b + 1, (k + 1) % 2)

            drain(b, k)
            compute(b, k)
        return 0

    lax.fori_loop(0, NBLK_A // 2, pair_body, 0)
    if NBLK_A % 2:
        b_last = NBLK_A - 1
        drain(b_last, b_last % 2)
        compute(b_last, b_last % 2)

    def drain_scatter(bb, _):
        pltpu.make_async_copy(ex_v.at[bb], shared_seg.at[sidx_v.at[bb]],
                              sem_sc).wait()
        return 0
    lax.fori_loop(0, NBLK_A, drain_scatter, 0)

    pltpu.sync_copy(ex_v, ex_hbm.at[wid])
    pltpu.sync_copy(cost_v, cost_hbm.at[wid])

    plsc.subcore_barrier()

    @pl.when(sid == 0)
    def _():
        for off, ln in ((0, 2048), (2048, 2048), (4096, 2048), (6144, 2048),
                        (8192, 1808)):
            pltpu.sync_copy(shared_seg.at[pl.ds(off, ln)],
                            zeros_v.at[pl.ds(0, ln)])
            pltpu.sync_copy(zeros_v.at[pl.ds(0, ln)],
                            partials_hbm.at[cid, pl.ds(off, ln)])


def _pass_b(partials_hbm, sidx_hbm, ex_hbm, w_hbm,
            seg_v, tmp_v, sidx_v, ex_v, w_v):
    cid = lax.axis_index("c")
    sid = lax.axis_index("s")
    wid = sid * NC + cid

    pltpu.sync_copy(partials_hbm.at[0], seg_v)
    pltpu.sync_copy(partials_hbm.at[1], tmp_v)

    def addp(i, _):
        sl = pl.ds(i * 16, 16)
        seg_v[sl] = seg_v[sl] + tmp_v[sl]
        return 0
    lax.fori_loop(0, N_S // 16, addp, 0)

    def body(b, _):
        base = wid * EPT + b * KB
        pltpu.sync_copy(sidx_hbm.at[pl.ds(base, KB)], sidx_v)
        pltpu.sync_copy(ex_hbm.at[pl.ds(base, KB)], ex_v)

        def gbody(g, _):
            sl = pl.ds(g * 16, 16)
            si = sidx_v[sl]
            denom = plsc.load_gather(seg_v, [si])
            w_v[sl] = ex_v[sl] / denom
            return 0
        lax.fori_loop(0, KB // 16, gbody, 0)

        pltpu.sync_copy(w_v, w_hbm.at[pl.ds(base, KB)])
        return 0

    lax.fori_loop(0, NBLK_B, body, 0)


def _pack_tbl(t, n):
    return lax.bitcast_convert_type(
        t.astype(_BF16).reshape(n, W, 2), _I32)


def kernel(embeddings_s, embeddings_a, edge_index_sa, log_temperature,
           W1, b1, W2, b2):
    s_idx = edge_index_sa[0].astype(_I32)
    a_idx = edge_index_sa[1].astype(_I32)

    t_s, t_a = pl.pallas_call(
        _tc_tables,
        out_shape=[
            jax.ShapeDtypeStruct((N_S, 2 * D), _F32),
            jax.ShapeDtypeStruct((N_A, 2 * D), _F32),
        ],
    )(embeddings_s, embeddings_a, W1, b1)

    ts_packed = _pack_tbl(t_s, N_S)
    ta_packed = _pack_tbl(t_a, N_A)

    sidx3 = s_idx.reshape(NW, NBLK_A, KA)
    aidx3 = a_idx.reshape(NW, NBLK_A, KA)

    inv_temp = jnp.exp(-log_temperature).reshape(1)
    misc = jnp.concatenate([b2.reshape(1).astype(_F32), inv_temp.astype(_F32),
                            jnp.zeros((14,), _F32)])
    w2_flat = W2.reshape(H).astype(_F32)

    mesh = plsc.VectorSubcoreMesh(core_axis_name="c", subcore_axis_name="s")
    sc_params = pltpu.CompilerParams(use_tc_tiling_on_sc=False,
                                     needs_layout_passes=False)

    pass_a = pl.kernel(
        _pass_a,
        out_type=(
            jax.ShapeDtypeStruct((NW, NBLK_A, KA), _F32),
            jax.ShapeDtypeStruct((NW, NBLK_A, KA), _F32),
            jax.ShapeDtypeStruct((NC, N_S), _F32),
        ),
        mesh=mesh,
        scratch_types=[
            pltpu.VMEM((KA, W), _I32),
            pltpu.VMEM((KA, W), _I32),
            pltpu.VMEM((KA, W), _I32),
            pltpu.VMEM((KA, W), _I32),
            pltpu.VMEM((NBLK_A, KA), _I32),
            pltpu.VMEM((NBLK_A, KA), _I32),
            pltpu.VMEM((NBLK_A, KA), _F32),
            pltpu.VMEM((NBLK_A, KA), _F32),
            pltpu.VMEM((H,), _F32),
            pltpu.VMEM((16,), _F32),
            pltpu.VMEM((2048,), _F32),
            pltpu.VMEM_SHARED((N_S,), _F32),
            pltpu.SemaphoreType.DMA,
            pltpu.SemaphoreType.DMA,
            pltpu.SemaphoreType.DMA,
        ],
        compiler_params=sc_params,
    )
    ex3, cost3, partials = pass_a(ts_packed, ta_packed, sidx3, aidx3,
                                  w2_flat, misc)
    ex = ex3.reshape(E)
    edge_costs = cost3.reshape(E)

    pass_b = pl.kernel(
        _pass_b,
        out_type=jax.ShapeDtypeStruct((E,), _F32),
        mesh=mesh,
        scratch_types=[
            pltpu.VMEM((N_S,), _F32),
            pltpu.VMEM((N_S,), _F32),
            pltpu.VMEM((KB,), _I32),
            pltpu.VMEM((KB,), _F32),
            pltpu.VMEM((KB,), _F32),
        ],
        compiler_params=sc_params,
    )
    edge_weights = pass_b(partials, s_idx, ex)

    return (edge_weights, edge_costs)


# speedup vs baseline: 1.0114x; 1.0114x over previous
import jax
import jax.numpy as jnp
from jax import lax
from jax.experimental import pallas as pl
from jax.experimental.pallas import tpu as pltpu
from jax.experimental.pallas import tpu_sc as plsc

N_S = 10000
N_A = 10000
E = 320000
D = 128
H = 128

NC = 2
NSUB = 16
NW = NC * NSUB
EPT = E // NW

W = D

KA = 80
NBLK_A = EPT // KA
KB = 2000
NBLK_B = EPT // KB
GRP = KA // 16

_F32 = jnp.float32
_I32 = jnp.int32
_BF16 = jnp.bfloat16
_ILV = None
_ILV = plsc.PackFormat.INTERLEAVED


def _tc_tables(es_ref, ea_ref, w1_ref, b1_ref, ts_ref, ta_ref):
    es = es_ref[...]
    ea = ea_ref[...]
    w1a = w1_ref[:D, :]
    w1b = w1_ref[D:, :]
    ts_ref[:, :D] = es
    ts_ref[:, D:] = jnp.dot(es, w1a, preferred_element_type=_F32) + b1_ref[...]
    ta_ref[:, :D] = ea
    ta_ref[:, D:] = jnp.dot(ea, w1b, preferred_element_type=_F32)


def _unpack2(word_vec):
    return plsc.unpack(plsc.bitcast(word_vec, _BF16), format=_ILV)


def _pass_a(ts_hbm, ta_hbm, sidx3_hbm, aidx3_hbm, w2_hbm, misc_hbm,
            ex_hbm, cost_hbm, partials_hbm,
            tsw0, tsw1, taw0, taw1, sidx_v, aidx_v, ex_v, cost_v,
            w2_v, misc_v, zeros_v, shared_seg, sem0, sem1, sem_sc):
    cid = lax.axis_index("c")
    sid = lax.axis_index("s")
    wid = sid * NC + cid

    pltpu.sync_copy(w2_hbm, w2_v)
    pltpu.sync_copy(misc_hbm, misc_v)
    pltpu.sync_copy(sidx3_hbm.at[wid], sidx_v)
    pltpu.sync_copy(aidx3_hbm.at[wid], aidx_v)


    def _zb(i, _):
        zeros_v[pl.ds(i * 16, 16)] = jnp.zeros((16,), _F32)
        return 0
    lax.fori_loop(0, 2048 // 16, _zb, 0)

    @pl.when(sid == 0)
    def _():
        for off, ln in ((0, 2048), (2048, 2048), (4096, 2048), (6144, 2048),
                        (8192, 1808)):
            pltpu.sync_copy(zeros_v.at[pl.ds(0, ln)],
                            shared_seg.at[pl.ds(off, ln)])

    plsc.subcore_barrier()

    iota16 = lax.iota(_I32, 16)
    z16 = jnp.zeros((16,), _F32)
    mv = misc_v[...]
    b2 = mv[0]
    invt = mv[1]

    tsbufs = (tsw0, tsw1)
    tabufs = (taw0, taw1)
    sems = (sem0, sem1)

    HK = KA // 2

    def fire(b, k):
        for tbl, idx, buf in ((ts_hbm, sidx_v, tsbufs[k]),
                              (ta_hbm, aidx_v, tabufs[k])):
            pltpu.async_copy(tbl.at[idx.at[b, pl.ds(0, HK)]],
                             buf.at[pl.ds(0, HK)], sems[k])
            pltpu.async_copy(tbl.at[idx.at[b, pl.ds(HK, HK)]],
                             buf.at[pl.ds(HK, HK)], sems[k])

    def drain(b, k):
        for tbl, idx, buf in ((ts_hbm, sidx_v, tsbufs[k]),
                              (ta_hbm, aidx_v, tabufs[k])):
            pltpu.make_async_copy(tbl.at[idx.at[b, pl.ds(0, HK)]],
                                  buf.at[pl.ds(0, HK)], sems[k]).wait()
            pltpu.make_async_copy(tbl.at[idx.at[b, pl.ds(HK, HK)]],
                                  buf.at[pl.ds(HK, HK)], sems[k]).wait()

    def compute(b, k):
        tsw = tsbufs[k]
        taw = tabufs[k]

        def gbody(g, _):
            rows = iota16 + g * 16
            zb = jnp.zeros((32,), _BF16)

            def jbody(jj, carry):
                ss = list(carry[0])
                dt = list(carry[1])
                w2a = w2_v[pl.ds(jj * 32, 16)]
                w2b = w2_v[pl.ds(jj * 32 + 16, 16)]
                for k2 in range(16):
                    lane = k2 % 4
                    colw = jnp.full((16,), jj * 16 + k2, _I32)
                    gts = plsc.bitcast(plsc.load_gather(tsw, [rows, colw]),
                                       _BF16)
                    gta = plsc.bitcast(plsc.load_gather(taw, [rows, colw]),
                                       _BF16)
                    d0, d1 = plsc.unpack(gts - gta, format=_ILV)
                    ss[lane] = ss[lane] + d0 * d0 + d1 * d1
                    colp = colw + (W // 2)
                    pts = plsc.bitcast(plsc.load_gather(tsw, [rows, colp]),
                                       _BF16)
                    pta = plsc.bitcast(plsc.load_gather(taw, [rows, colp]),
                                       _BF16)
                    h0, h1 = plsc.unpack(jnp.maximum(pts + pta, zb),
                                         format=_ILV)
                    if k2 < 8:
                        w20 = w2a[2 * k2]
                        w21 = w2a[2 * k2 + 1]
                    else:
                        w20 = w2b[2 * k2 - 16]
                        w21 = w2b[2 * k2 - 15]
                    dt[lane] = (dt[lane] + h0 * jnp.full((16,), w20, _F32)
                                + h1 * jnp.full((16,), w21, _F32))
                return tuple(ss), tuple(dt)

            z4 = (z16, z16, z16, z16)
            ss4, dt4 = plsc.parallel_loop(
                0, W // 32, unroll=4, carry=(z4, z4))(jbody)
            acc_ss = (ss4[0] + ss4[1]) + (ss4[2] + ss4[3])
            acc_dot = (dt4[0] + dt4[1]) + (dt4[2] + dt4[3])

            ibits = plsc.bitcast(acc_ss, _I32)
            y = plsc.bitcast(jnp.int32(0x5F3759DF) - (ibits >> 1), _F32)
            for _ in range(3):
                y = y * (1.5 - 0.5 * acc_ss * y * y)
            cost = acc_ss * y

            gate = 1.0 / (1.0 + jnp.exp(-(acc_dot + b2)))
            costg = cost * gate
            ex = jnp.exp(-costg * invt)
            cost_v[b, pl.ds(g * 16, 16)] = costg
            ex_v[b, pl.ds(g * 16, 16)] = ex
            return 0

        lax.fori_loop(0, GRP, gbody, 0)
        pltpu.async_copy(ex_v.at[b], shared_seg.at[sidx_v.at[b]], sem_sc,
                         add=True)

    fire(0, 0)

    def pair_body(p, _):
        for k in (0, 1):
            b = p * 2 + k

            @pl.when(b + 1 < NBLK_A)
            def _():
                fire(b + 1, (k + 1) % 2)

            drain(b, k)
            compute(b, k)
        return 0

    lax.fori_loop(0, NBLK_A // 2, pair_body, 0)
    if NBLK_A % 2:
        b_last = NBLK_A - 1
        drain(b_last, b_last % 2)
        compute(b_last, b_last % 2)

    def drain_scatter(bb, _):
        pltpu.make_async_copy(ex_v.at[bb], shared_seg.at[sidx_v.at[bb]],
                              sem_sc).wait()
        return 0
    lax.fori_loop(0, NBLK_A, drain_scatter, 0)

    pltpu.sync_copy(ex_v, ex_hbm.at[wid])
    pltpu.sync_copy(cost_v, cost_hbm.at[wid])

    plsc.subcore_barrier()

    @pl.when(sid == 0)
    def _():
        for off, ln in ((0, 2048), (2048, 2048), (4096, 2048), (6144, 2048),
                        (8192, 1808)):
            pltpu.sync_copy(shared_seg.at[pl.ds(off, ln)],
                            zeros_v.at[pl.ds(0, ln)])
            pltpu.sync_copy(zeros_v.at[pl.ds(0, ln)],
                            partials_hbm.at[cid, pl.ds(off, ln)])


def _pass_b(partials_hbm, sidx_hbm, ex_hbm, w_hbm,
            seg_v, tmp_v, sidx_v, ex_v, w_v):
    cid = lax.axis_index("c")
    sid = lax.axis_index("s")
    wid = sid * NC + cid

    pltpu.sync_copy(partials_hbm.at[0], seg_v)
    pltpu.sync_copy(partials_hbm.at[1], tmp_v)

    def addp(i, _):
        sl = pl.ds(i * 16, 16)
        seg_v[sl] = seg_v[sl] + tmp_v[sl]
        return 0
    lax.fori_loop(0, N_S // 16, addp, 0)

    def body(b, _):
        base = wid * EPT + b * KB
        pltpu.sync_copy(sidx_hbm.at[pl.ds(base, KB)], sidx_v)
        pltpu.sync_copy(ex_hbm.at[pl.ds(base, KB)], ex_v)

        def gbody(g, _):
            sl = pl.ds(g * 16, 16)
            si = sidx_v[sl]
            denom = plsc.load_gather(seg_v, [si])
            w_v[sl] = ex_v[sl] / denom
            return 0
        lax.fori_loop(0, KB // 16, gbody, 0)

        pltpu.sync_copy(w_v, w_hbm.at[pl.ds(base, KB)])
        return 0

    lax.fori_loop(0, NBLK_B, body, 0)


def _pack_tbl(t, n):
    return lax.bitcast_convert_type(
        t.astype(_BF16).reshape(n, W, 2), _I32)


def kernel(embeddings_s, embeddings_a, edge_index_sa, log_temperature,
           W1, b1, W2, b2):
    s_idx = edge_index_sa[0].astype(_I32)
    a_idx = edge_index_sa[1].astype(_I32)

    t_s, t_a = pl.pallas_call(
        _tc_tables,
        out_shape=[
            jax.ShapeDtypeStruct((N_S, 2 * D), _F32),
            jax.ShapeDtypeStruct((N_A, 2 * D), _F32),
        ],
    )(embeddings_s, embeddings_a, W1, b1)

    ts_packed = _pack_tbl(t_s, N_S)
    ta_packed = _pack_tbl(t_a, N_A)

    sidx3 = s_idx.reshape(NW, NBLK_A, KA)
    aidx3 = a_idx.reshape(NW, NBLK_A, KA)

    inv_temp = jnp.exp(-log_temperature).reshape(1)
    misc = jnp.concatenate([b2.reshape(1).astype(_F32), inv_temp.astype(_F32),
                            jnp.zeros((14,), _F32)])
    w2_flat = W2.reshape(H).astype(_F32)

    mesh = plsc.VectorSubcoreMesh(core_axis_name="c", subcore_axis_name="s")
    sc_params = pltpu.CompilerParams(use_tc_tiling_on_sc=False,
                                     needs_layout_passes=False)

    pass_a = pl.kernel(
        _pass_a,
        out_type=(
            jax.ShapeDtypeStruct((NW, NBLK_A, KA), _F32),
            jax.ShapeDtypeStruct((NW, NBLK_A, KA), _F32),
            jax.ShapeDtypeStruct((NC, N_S), _F32),
        ),
        mesh=mesh,
        scratch_types=[
            pltpu.VMEM((KA, W), _I32),
            pltpu.VMEM((KA, W), _I32),
            pltpu.VMEM((KA, W), _I32),
            pltpu.VMEM((KA, W), _I32),
            pltpu.VMEM((NBLK_A, KA), _I32),
            pltpu.VMEM((NBLK_A, KA), _I32),
            pltpu.VMEM((NBLK_A, KA), _F32),
            pltpu.VMEM((NBLK_A, KA), _F32),
            pltpu.VMEM((H,), _F32),
            pltpu.VMEM((16,), _F32),
            pltpu.VMEM((2048,), _F32),
            pltpu.VMEM_SHARED((N_S,), _F32),
            pltpu.SemaphoreType.DMA,
            pltpu.SemaphoreType.DMA,
            pltpu.SemaphoreType.DMA,
        ],
        compiler_params=sc_params,
    )
    ex3, cost3, partials = pass_a(ts_packed, ta_packed, sidx3, aidx3,
                                  w2_flat, misc)
    ex = ex3.reshape(E)
    edge_costs = cost3.reshape(E)

    pass_b = pl.kernel(
        _pass_b,
        out_type=jax.ShapeDtypeStruct((E,), _F32),
        mesh=mesh,
        scratch_types=[
            pltpu.VMEM((N_S,), _F32),
            pltpu.VMEM((N_S,), _F32),
            pltpu.VMEM((KB,), _I32),
            pltpu.VMEM((KB,), _F32),
            pltpu.VMEM((KB,), _F32),
        ],
        compiler_params=sc_params,
    )
    edge_weights = pass_b(partials, s_idx, ex)

    return (edge_weights, edge_costs)


# speedup vs baseline: 1.5485x; 1.5311x over previous
import jax
import jax.numpy as jnp
from jax import lax
from jax.experimental import pallas as pl
from jax.experimental.pallas import tpu as pltpu
from jax.experimental.pallas import tpu_sc as plsc

N_S = 10000
N_A = 10000
E = 320000
D = 128
H = 128

NC = 2
NSUB = 16
NW = NC * NSUB
EPT = E // NW

W = D
WPAD = 137

KA = 80
NBLK_A = EPT // KA
KB = 2000
NBLK_B = EPT // KB
GRP = KA // 16

_F32 = jnp.float32
_I32 = jnp.int32
_BF16 = jnp.bfloat16
_ILV = None
_ILV = plsc.PackFormat.INTERLEAVED


def _tc_tables(es_ref, ea_ref, w1_ref, b1_ref, ts_ref, ta_ref):
    es = es_ref[...]
    ea = ea_ref[...]
    w1a = w1_ref[:D, :]
    w1b = w1_ref[D:, :]
    ts_ref[:, :D] = es
    ts_ref[:, D:] = jnp.dot(es, w1a, preferred_element_type=_F32) + b1_ref[...]
    ta_ref[:, :D] = ea
    ta_ref[:, D:] = jnp.dot(ea, w1b, preferred_element_type=_F32)


def _unpack2(word_vec):
    return plsc.unpack(plsc.bitcast(word_vec, _BF16), format=_ILV)


def _pass_a(ts_hbm, ta_hbm, sidx3_hbm, aidx3_hbm, w2_hbm, misc_hbm,
            ex_hbm, cost_hbm, partials_hbm,
            tsw0, tsw1, taw0, taw1, sidx_v, aidx_v, ex_v, cost_v,
            w2_v, misc_v, zeros_v, shared_seg, sem0, sem1, sem_sc):
    cid = lax.axis_index("c")
    sid = lax.axis_index("s")
    wid = sid * NC + cid

    pltpu.sync_copy(w2_hbm, w2_v)
    pltpu.sync_copy(misc_hbm, misc_v)
    pltpu.sync_copy(sidx3_hbm.at[wid], sidx_v)
    pltpu.sync_copy(aidx3_hbm.at[wid], aidx_v)


    def _zb(i, _):
        zeros_v[pl.ds(i * 16, 16)] = jnp.zeros((16,), _F32)
        return 0
    lax.fori_loop(0, 2048 // 16, _zb, 0)

    @pl.when(sid == 0)
    def _():
        for off, ln in ((0, 2048), (2048, 2048), (4096, 2048), (6144, 2048),
                        (8192, 1808)):
            pltpu.sync_copy(zeros_v.at[pl.ds(0, ln)],
                            shared_seg.at[pl.ds(off, ln)])

    plsc.subcore_barrier()

    iota16 = lax.iota(_I32, 16)
    z16 = jnp.zeros((16,), _F32)
    mv = misc_v[...]
    b2 = mv[0]
    invt = mv[1]

    tsbufs = (tsw0, tsw1)
    tabufs = (taw0, taw1)
    sems = (sem0, sem1)

    HK = KA // 2

    def fire(b, k):
        for tbl, idx, buf in ((ts_hbm, sidx_v, tsbufs[k]),
                              (ta_hbm, aidx_v, tabufs[k])):
            pltpu.async_copy(tbl.at[idx.at[b, pl.ds(0, HK)]],
                             buf.at[pl.ds(0, HK)], sems[k])
            pltpu.async_copy(tbl.at[idx.at[b, pl.ds(HK, HK)]],
                             buf.at[pl.ds(HK, HK)], sems[k])

    def drain(b, k):
        for tbl, idx, buf in ((ts_hbm, sidx_v, tsbufs[k]),
                              (ta_hbm, aidx_v, tabufs[k])):
            pltpu.make_async_copy(tbl.at[idx.at[b, pl.ds(0, HK)]],
                                  buf.at[pl.ds(0, HK)], sems[k]).wait()
            pltpu.make_async_copy(tbl.at[idx.at[b, pl.ds(HK, HK)]],
                                  buf.at[pl.ds(HK, HK)], sems[k]).wait()

    def compute(b, k):
        tsw = tsbufs[k]
        taw = tabufs[k]

        def gbody(g, _):
            rows = iota16 + g * 16
            zb = jnp.zeros((32,), _BF16)

            def jbody(jj, carry):
                ss = list(carry[0])
                dt = list(carry[1])
                w2a = w2_v[pl.ds(jj * 32, 16)]
                w2b = w2_v[pl.ds(jj * 32 + 16, 16)]
                for k2 in range(16):
                    lane = k2 % 4
                    colw = jnp.full((16,), jj * 16 + k2, _I32)
                    gts = plsc.bitcast(plsc.load_gather(tsw, [rows, colw]),
                                       _BF16)
                    gta = plsc.bitcast(plsc.load_gather(taw, [rows, colw]),
                                       _BF16)
                    d0, d1 = plsc.unpack(gts - gta, format=_ILV)
                    ss[lane] = ss[lane] + d0 * d0 + d1 * d1
                    colp = colw + (W // 2)
                    pts = plsc.bitcast(plsc.load_gather(tsw, [rows, colp]),
                                       _BF16)
                    pta = plsc.bitcast(plsc.load_gather(taw, [rows, colp]),
                                       _BF16)
                    h0, h1 = plsc.unpack(jnp.maximum(pts + pta, zb),
                                         format=_ILV)
                    if k2 < 8:
                        w20 = w2a[2 * k2]
                        w21 = w2a[2 * k2 + 1]
                    else:
                        w20 = w2b[2 * k2 - 16]
                        w21 = w2b[2 * k2 - 15]
                    dt[lane] = (dt[lane] + h0 * jnp.full((16,), w20, _F32)
                                + h1 * jnp.full((16,), w21, _F32))
                return tuple(ss), tuple(dt)

            z4 = (z16, z16, z16, z16)
            ss4, dt4 = plsc.parallel_loop(
                0, W // 32, unroll=4, carry=(z4, z4))(jbody)
            acc_ss = (ss4[0] + ss4[1]) + (ss4[2] + ss4[3])
            acc_dot = (dt4[0] + dt4[1]) + (dt4[2] + dt4[3])

            ibits = plsc.bitcast(acc_ss, _I32)
            y = plsc.bitcast(jnp.int32(0x5F3759DF) - (ibits >> 1), _F32)
            for _ in range(3):
                y = y * (1.5 - 0.5 * acc_ss * y * y)
            cost = acc_ss * y

            gate = 1.0 / (1.0 + jnp.exp(-(acc_dot + b2)))
            costg = cost * gate
            ex = jnp.exp(-costg * invt)
            cost_v[b, pl.ds(g * 16, 16)] = costg
            ex_v[b, pl.ds(g * 16, 16)] = ex
            return 0

        lax.fori_loop(0, GRP, gbody, 0)
        pltpu.async_copy(ex_v.at[b], shared_seg.at[sidx_v.at[b]], sem_sc,
                         add=True)

    fire(0, 0)

    def pair_body(p, _):
        for k in (0, 1):
            b = p * 2 + k

            @pl.when(b + 1 < NBLK_A)
            def _():
                fire(b + 1, (k + 1) % 2)

            drain(b, k)
            compute(b, k)
        return 0

    lax.fori_loop(0, NBLK_A // 2, pair_body, 0)
    if NBLK_A % 2:
        b_last = NBLK_A - 1
        drain(b_last, b_last % 2)
        compute(b_last, b_last % 2)

    def drain_scatter(bb, _):
        pltpu.make_async_copy(ex_v.at[bb], shared_seg.at[sidx_v.at[bb]],
                              sem_sc).wait()
        return 0
    lax.fori_loop(0, NBLK_A, drain_scatter, 0)

    pltpu.sync_copy(ex_v, ex_hbm.at[wid])
    pltpu.sync_copy(cost_v, cost_hbm.at[wid])

    plsc.subcore_barrier()

    @pl.when(sid == 0)
    def _():
        for off, ln in ((0, 2048), (2048, 2048), (4096, 2048), (6144, 2048),
                        (8192, 1808)):
            pltpu.sync_copy(shared_seg.at[pl.ds(off, ln)],
                            zeros_v.at[pl.ds(0, ln)])
            pltpu.sync_copy(zeros_v.at[pl.ds(0, ln)],
                            partials_hbm.at[cid, pl.ds(off, ln)])


def _pass_b(partials_hbm, sidx_hbm, ex_hbm, w_hbm,
            seg_v, tmp_v, sidx_v, ex_v, w_v):
    cid = lax.axis_index("c")
    sid = lax.axis_index("s")
    wid = sid * NC + cid

    pltpu.sync_copy(partials_hbm.at[0], seg_v)
    pltpu.sync_copy(partials_hbm.at[1], tmp_v)

    def addp(i, _):
        sl = pl.ds(i * 16, 16)
        seg_v[sl] = seg_v[sl] + tmp_v[sl]
        return 0
    lax.fori_loop(0, N_S // 16, addp, 0)

    def body(b, _):
        base = wid * EPT + b * KB
        pltpu.sync_copy(sidx_hbm.at[pl.ds(base, KB)], sidx_v)
        pltpu.sync_copy(ex_hbm.at[pl.ds(base, KB)], ex_v)

        def gbody(g, _):
            sl = pl.ds(g * 16, 16)
            si = sidx_v[sl]
            denom = plsc.load_gather(seg_v, [si])
            w_v[sl] = ex_v[sl] / denom
            return 0
        lax.fori_loop(0, KB // 16, gbody, 0)

        pltpu.sync_copy(w_v, w_hbm.at[pl.ds(base, KB)])
        return 0

    lax.fori_loop(0, NBLK_B, body, 0)


def _pack_tbl(t, n):
    packed = lax.bitcast_convert_type(t.astype(_BF16).reshape(n, W, 2), _I32)
    return jnp.pad(packed, ((0, 0), (0, WPAD - W)))


def kernel(embeddings_s, embeddings_a, edge_index_sa, log_temperature,
           W1, b1, W2, b2):
    s_idx = edge_index_sa[0].astype(_I32)
    a_idx = edge_index_sa[1].astype(_I32)

    t_s, t_a = pl.pallas_call(
        _tc_tables,
        out_shape=[
            jax.ShapeDtypeStruct((N_S, 2 * D), _F32),
            jax.ShapeDtypeStruct((N_A, 2 * D), _F32),
        ],
    )(embeddings_s, embeddings_a, W1, b1)

    ts_packed = _pack_tbl(t_s, N_S)
    ta_packed = _pack_tbl(t_a, N_A)

    sidx3 = s_idx.reshape(NW, NBLK_A, KA)
    aidx3 = a_idx.reshape(NW, NBLK_A, KA)

    inv_temp = jnp.exp(-log_temperature).reshape(1)
    misc = jnp.concatenate([b2.reshape(1).astype(_F32), inv_temp.astype(_F32),
                            jnp.zeros((14,), _F32)])
    w2_flat = W2.reshape(H).astype(_F32)

    mesh = plsc.VectorSubcoreMesh(core_axis_name="c", subcore_axis_name="s")
    sc_params = pltpu.CompilerParams(use_tc_tiling_on_sc=False,
                                     needs_layout_passes=False)

    pass_a = pl.kernel(
        _pass_a,
        out_type=(
            jax.ShapeDtypeStruct((NW, NBLK_A, KA), _F32),
            jax.ShapeDtypeStruct((NW, NBLK_A, KA), _F32),
            jax.ShapeDtypeStruct((NC, N_S), _F32),
        ),
        mesh=mesh,
        scratch_types=[
            pltpu.VMEM((KA, WPAD), _I32),
            pltpu.VMEM((KA, WPAD), _I32),
            pltpu.VMEM((KA, WPAD), _I32),
            pltpu.VMEM((KA, WPAD), _I32),
            pltpu.VMEM((NBLK_A, KA), _I32),
            pltpu.VMEM((NBLK_A, KA), _I32),
            pltpu.VMEM((NBLK_A, KA), _F32),
            pltpu.VMEM((NBLK_A, KA), _F32),
            pltpu.VMEM((H,), _F32),
            pltpu.VMEM((16,), _F32),
            pltpu.VMEM((2048,), _F32),
            pltpu.VMEM_SHARED((N_S,), _F32),
            pltpu.SemaphoreType.DMA,
            pltpu.SemaphoreType.DMA,
            pltpu.SemaphoreType.DMA,
        ],
        compiler_params=sc_params,
    )
    ex3, cost3, partials = pass_a(ts_packed, ta_packed, sidx3, aidx3,
                                  w2_flat, misc)
    ex = ex3.reshape(E)
    edge_costs = cost3.reshape(E)

    pass_b = pl.kernel(
        _pass_b,
        out_type=jax.ShapeDtypeStruct((E,), _F32),
        mesh=mesh,
        scratch_types=[
            pltpu.VMEM((N_S,), _F32),
            pltpu.VMEM((N_S,), _F32),
            pltpu.VMEM((KB,), _I32),
            pltpu.VMEM((KB,), _F32),
            pltpu.VMEM((KB,), _F32),
        ],
        compiler_params=sc_params,
    )
    edge_weights = pass_b(partials, s_idx, ex)

    return (edge_weights, edge_costs)
